# baseline (speedup 1.0000x reference)
import functools

import jax
import jax.numpy as jnp
from jax import lax
from jax.experimental import pallas as pl
from jax.experimental.pallas import tpu as pltpu
from jax.experimental.pallas import tpu_sc as plsc

_LN2 = 0.6931471805599453


def _vsqrt(v):
    i = plsc.bitcast(v, jnp.int32)
    y = plsc.bitcast(jnp.int32(0x5F3759DF) - (i >> 1), jnp.float32)
    for _ in range(3):
        y = y * (1.5 - 0.5 * v * y * y)
    return v * y


def _vln(r):
    i = plsc.bitcast(r, jnp.int32)
    ex = ((i >> 23) & 0xFF) - 127
    f = plsc.bitcast((i & 0x007FFFFF) | 0x3F800000, jnp.float32)
    y = ex.astype(jnp.float32) * _LN2 + (f - 1.0)
    for _ in range(3):
        y = y - 1.0 + r * jnp.exp(-y)
    return y


def kernel(z_all, pre_knn_indices, cell_indices):
    n_cells, dim = z_all.shape
    batch = cell_indices.shape[0]
    k = pre_knn_indices.shape[1]
    kp = 32
    spe = 2 * kp

    info = plsc.get_sparse_core_info()
    nw = info.num_cores * info.num_subcores
    bpw = batch // nw
    nent = bpw * spe

    tblk = 8192
    nblk = -(-n_cells // tblk)
    cap = 1024
    rnd = 512

    cell32 = cell_indices.astype(jnp.int32)
    knn32 = pre_knn_indices.astype(jnp.int32)
    neg = jax.random.randint(jax.random.key(1234), (batch, k), 0, n_cells,
                             dtype=jnp.int32)

    mesh = plsc.VectorSubcoreMesh(core_axis_name="c", subcore_axis_name="s")

    @functools.partial(
        pl.kernel,
        out_type=jax.ShapeDtypeStruct((nw, 16), jnp.float32),
        mesh=mesh,
        compiler_params=pltpu.CompilerParams(needs_layout_passes=False,
                                             use_tc_tiling_on_sc=False),
        scratch_types=[
            pltpu.VMEM_SHARED((tblk, dim), jnp.float32),
            pltpu.VMEM((bpw,), jnp.int32),
            pltpu.VMEM((bpw, dim), jnp.float32),
            pltpu.VMEM((bpw, k), jnp.int32),
            pltpu.VMEM((bpw, k), jnp.int32),
            pltpu.VMEM((nent,), jnp.int32),
            pltpu.VMEM((cap,), jnp.int32),
            pltpu.VMEM((cap,), jnp.int32),
            pltpu.VMEM((rnd, dim), jnp.float32),
            pltpu.VMEM((nent,), jnp.float32),
            pltpu.VMEM((16,), jnp.float32),
            pltpu.SemaphoreType.DMA,
            pltpu.SemaphoreType.DMA,
            pltpu.SemaphoreType.DMA,
        ],
    )
    def sc_kernel(z_hbm, knn_hbm, neg_hbm, cell_hbm, out_hbm,
                  blk, cidx, zi, nnp, ngs, cmb, cpos, cslot, crows, distb,
                  accv, sem, sem2, sem3):
        cid = lax.axis_index("c")
        sid = lax.axis_index("s")
        wid = sid * info.num_cores + cid
        base = pl.multiple_of(wid * bpw, bpw)
        lanes = lax.iota(jnp.int32, 16)
        zero = jnp.zeros((16,), jnp.float32)
        izero = jnp.zeros((16,), jnp.int32)

        pltpu.sync_copy(cell_hbm.at[pl.ds(base, bpw)], cidx)
        cp_zi = pltpu.make_async_copy(z_hbm.at[cidx], zi, sem2)
        cp_nn = pltpu.make_async_copy(knn_hbm.at[cidx], nnp, sem3)
        cp_zi.start()
        cp_nn.start()
        pltpu.sync_copy(neg_hbm.at[pl.ds(base, bpw)], ngs)

        def stage(p):
            lo = p * tblk
            rows = min(n_cells - lo, tblk)
            ns = info.num_subcores
            cpt = -(-rows // ns // 8) * 8
            off = pl.multiple_of(jnp.minimum(sid * cpt, rows - cpt), 8)
            cp = pltpu.make_async_copy(
                z_hbm.at[pl.ds(lo + off, cpt)], blk.at[pl.ds(off, cpt)], sem)
            cp.start()
            cp.wait()

        stage(0)
        cp_nn.wait()

        def rp_body(e, carry):
            cmb[pl.ds(e * spe, 16)] = nnp[e, pl.ds(0, 16)]
            cmb[pl.ds(e * spe + 16, 16)] = nnp[e, pl.ds(k - 16, 16)]
            cmb[pl.ds(e * spe + kp, 16)] = ngs[e, pl.ds(0, 16)]
            cmb[pl.ds(e * spe + kp + 16, 16)] = ngs[e, pl.ds(k - 16, 16)]
            return carry

        lax.fori_loop(0, bpw, rp_body, 0)

        def init_body(i, carry):
            cpos[pl.ds(i * 16, 16)] = izero
            return carry

        lax.fori_loop(0, cap // 16, init_body, 0)
        cp_zi.wait()
        plsc.subcore_barrier()

        for p in range(nblk):
            lo = p * tblk
            hi = min(lo + tblk, n_cells)

            def scan_body(g, cntv):
                v = cmb[pl.ds(g * 16, 16)]
                m = (v >= lo) & (v < hi)
                cs = plsc.cumsum(m.astype(jnp.int32))
                wpos = cntv + cs - 1
                wm = m & (wpos < cap)
                plsc.store_scatter(cpos, [wpos], v - lo, mask=wm)
                plsc.store_scatter(cslot, [wpos], lanes + g * 16, mask=wm)
                return cntv + plsc.all_reduce_population_count(m)

            cntv = lax.fori_loop(0, nent // 16, scan_body, izero)
            cnt = cntv[0]
            nround = jnp.minimum((cnt + (rnd - 1)) // rnd, cap // rnd)

            def round_body(rr, acc_unused):
                r0 = pl.multiple_of(rr * rnd, rnd)
                cps = [pltpu.make_async_copy(
                    blk.at[cpos.at[pl.ds(r0 + u * 128, 128)]],
                    crows.at[pl.ds(u * 128, 128)], sem)
                    for u in range(rnd // 128)]
                for cp in cps:
                    cp.start()
                for cp in cps:
                    cp.wait()

                def group_body(g, carry):
                    i0 = r0 + g * 16
                    islot = cslot[pl.ds(i0, 16)]
                    ev = (islot >> 6) & (bpw - 1)
                    rows = lanes + g * 16

                    def dbody(qq, a):
                        for r in range(4):
                            cols = jnp.full((16,), qq * 4 + r, jnp.int32)
                            t = (plsc.load_gather(crows, [rows, cols])
                                 - plsc.load_gather(zi, [ev, cols]))
                            a = a + t * t
                        return a

                    ssq = lax.fori_loop(0, dim // 4, dbody, zero)
                    wm = (lanes + i0) < cnt
                    plsc.store_scatter(distb, [islot], ssq, mask=wm)
                    return carry

                lax.fori_loop(0, rnd // 16, group_body, 0)
                return acc_unused

            lax.fori_loop(0, nround, round_body, 0)
            plsc.subcore_barrier()
            if p + 1 < nblk:
                stage(p + 1)
                plsc.subcore_barrier()

        dup = kp - k

        def loss_body(e, acc):
            s0 = pl.multiple_of(e * spe, spe)
            d0 = _vsqrt(distb[pl.ds(s0, 16)])
            p1v = distb[pl.ds(s0 + 16, 16)]
            p1v = jnp.where(lanes >= dup, p1v, 1e30)
            d1 = _vsqrt(p1v)
            d2 = _vsqrt(distb[pl.ds(s0 + kp, 16)])
            q1v = distb[pl.ds(s0 + kp + 16, 16)]
            q1v = jnp.where(lanes >= dup, q1v, 1e30)
            d3 = _vsqrt(q1v)
            m = jnp.min(jnp.minimum(jnp.minimum(d0, d1),
                                    jnp.minimum(d2, d3)))
            mv = jnp.full((16,), m, jnp.float32)
            e0v = jnp.exp(mv - d0)
            e1v = jnp.exp(mv - d1)
            e2v = jnp.exp(mv - d2)
            e3v = jnp.exp(mv - d3)
            sp = jnp.full((16,), jnp.sum(e0v + e1v), jnp.float32)
            st = sp + jnp.full((16,), jnp.sum(e2v + e3v), jnp.float32)
            ratio = st / (sp + 1e-8 * st)
            return acc + _vln(ratio)

        acc = lax.fori_loop(0, bpw, loss_body, zero)
        accv[...] = acc
        pltpu.sync_copy(accv, out_hbm.at[wid])

    partial = sc_kernel(z_all, knn32, neg, cell32)
    return jnp.sum(partial) / (16.0 * batch)


# speedup vs baseline: 2.6038x; 2.6038x over previous
import functools

import jax
import jax.numpy as jnp
from jax import lax
from jax.experimental import pallas as pl
from jax.experimental.pallas import tpu as pltpu
from jax.experimental.pallas import tpu_sc as plsc

_LN2 = 0.6931471805599453


def _vsqrt(v):
    i = plsc.bitcast(v, jnp.int32)
    y = plsc.bitcast(jnp.int32(0x5F3759DF) - (i >> 1), jnp.float32)
    for _ in range(3):
        y = y * (1.5 - 0.5 * v * y * y)
    return v * y


def _vln(r):
    i = plsc.bitcast(r, jnp.int32)
    ex = ((i >> 23) & 0xFF) - 127
    f = plsc.bitcast((i & 0x007FFFFF) | 0x3F800000, jnp.float32)
    y = ex.astype(jnp.float32) * _LN2 + (f - 1.0)
    for _ in range(3):
        y = y - 1.0 + r * jnp.exp(-y)
    return y


def kernel(z_all, pre_knn_indices, cell_indices):
    n_cells, dim = z_all.shape
    batch = cell_indices.shape[0]
    k = pre_knn_indices.shape[1]
    kp = 32

    info = plsc.get_sparse_core_info()
    nw = info.num_cores * info.num_subcores
    bpw = batch // nw
    ch = 4
    nchunk = bpw // ch
    nidx = ch * kp

    cell32 = cell_indices.astype(jnp.int32)
    knn32 = pre_knn_indices.astype(jnp.int32)
    knn_pad = jnp.concatenate([knn32, knn32[:, :kp - k]], axis=1)
    neg = jax.random.randint(jax.random.key(1234), (batch, k), 0, n_cells,
                             dtype=jnp.int32)
    neg_flat = jnp.concatenate([neg, neg[:, :kp - k]],
                               axis=1).reshape(batch * kp)

    mesh = plsc.VectorSubcoreMesh(core_axis_name="c", subcore_axis_name="s")

    @functools.partial(
        pl.kernel,
        out_type=jax.ShapeDtypeStruct((nw, 16), jnp.float32),
        mesh=mesh,
        compiler_params=pltpu.CompilerParams(needs_layout_passes=False,
                                             use_tc_tiling_on_sc=False),
        scratch_types=[
            pltpu.VMEM((bpw,), jnp.int32),
            pltpu.VMEM((bpw, dim), jnp.float32),
            pltpu.VMEM((bpw, kp), jnp.int32),
            pltpu.VMEM((bpw * kp,), jnp.int32),
            pltpu.VMEM((bpw * kp,), jnp.int32),
            pltpu.VMEM((4, nidx, dim), jnp.float32),
            pltpu.VMEM((4, nidx, dim), jnp.float32),
            pltpu.VMEM((16,), jnp.float32),
            pltpu.SemaphoreType.DMA,
            pltpu.SemaphoreType.DMA,
            pltpu.SemaphoreType.DMA,
            pltpu.SemaphoreType.DMA,
            pltpu.SemaphoreType.DMA,
            pltpu.SemaphoreType.DMA,
            pltpu.SemaphoreType.DMA,
            pltpu.SemaphoreType.DMA,
            pltpu.SemaphoreType.DMA,
        ],
    )
    def sc_kernel(z_hbm, knn_hbm, neg_hbm, cell_hbm, out_hbm,
                  cidx, zi, nnp, nnf, ngf, posb, negb, accv, sem,
                  ps0, ps1, ps2, ps3, ns0, ns1, ns2, ns3):
        wid = lax.axis_index("s") * info.num_cores + lax.axis_index("c")
        base = pl.multiple_of(wid * bpw, bpw)
        pltpu.sync_copy(cell_hbm.at[pl.ds(base, bpw)], cidx)
        pltpu.async_copy(z_hbm.at[cidx], zi, sem).wait()
        pltpu.async_copy(knn_hbm.at[cidx], nnp, sem).wait()
        fbase = pl.multiple_of(wid * (bpw * kp), bpw * kp)
        pltpu.sync_copy(neg_hbm.at[pl.ds(fbase, bpw * kp)], ngf)

        def rp_body(e, carry):
            nnf[pl.ds(e * kp, 16)] = nnp[e, pl.ds(0, 16)]
            nnf[pl.ds(e * kp + 16, 16)] = nnp[e, pl.ds(16, 16)]
            return carry

        lax.fori_loop(0, bpw, rp_body, 0)

        lanes = lax.iota(jnp.int32, 16)
        zero = jnp.zeros((16,), jnp.float32)
        psems = [ps0, ps1, ps2, ps3]
        nsems = [ns0, ns1, ns2, ns3]
        nb = 4

        def cp_pos(c, b):
            i0 = pl.multiple_of(c * nidx, nidx)
            return pltpu.make_async_copy(
                z_hbm.at[nnf.at[pl.ds(i0, nidx)]], posb.at[b], psems[b])

        def cp_neg(c, b):
            i0 = pl.multiple_of(c * nidx, nidx)
            return pltpu.make_async_copy(
                z_hbm.at[ngf.at[pl.ds(i0, nidx)]], negb.at[b], nsems[b])

        for b in range(nb):
            cp_pos(b, b).start()
            cp_neg(b, b).start()

        dup = 16 - (kp - k)

        def compute_chunk(c, b, acc):
            bv = jnp.full((16,), b, jnp.int32)
            for ee in range(ch):
                e = c * ch + ee
                r0 = lanes + (ee * kp)
                r1 = r0 + 16

                def dbody(qq, accs):
                    a0, a1, a2, a3 = accs
                    zq = zi[e, pl.ds(qq * 16, 16)]
                    for r in range(16):
                        zv = zq[r]
                        cols = jnp.full((16,), qq * 16 + r, jnp.int32)
                        t0 = plsc.load_gather(posb, [bv, r0, cols]) - zv
                        t1 = plsc.load_gather(posb, [bv, r1, cols]) - zv
                        t2 = plsc.load_gather(negb, [bv, r0, cols]) - zv
                        t3 = plsc.load_gather(negb, [bv, r1, cols]) - zv
                        a0 = a0 + t0 * t0
                        a1 = a1 + t1 * t1
                        a2 = a2 + t2 * t2
                        a3 = a3 + t3 * t3
                    return (a0, a1, a2, a3)

                p0, p1, q0, q1 = lax.fori_loop(
                    0, dim // 16, dbody, (zero, zero, zero, zero))
                p1 = jnp.where(lanes < dup, p1, 1e30)
                q1 = jnp.where(lanes < dup, q1, 1e30)
                d0 = _vsqrt(p0)
                d1 = _vsqrt(p1)
                d2 = _vsqrt(q0)
                d3 = _vsqrt(q1)
                m = jnp.min(jnp.minimum(jnp.minimum(d0, d1),
                                        jnp.minimum(d2, d3)))
                mv = jnp.full((16,), m, jnp.float32)
                e0v = jnp.exp(mv - d0)
                e1v = jnp.exp(mv - d1)
                e2v = jnp.exp(mv - d2)
                e3v = jnp.exp(mv - d3)
                sp = jnp.full((16,), jnp.sum(e0v + e1v), jnp.float32)
                st = sp + jnp.full((16,), jnp.sum(e2v + e3v), jnp.float32)
                ratio = st / (sp + 1e-8 * st)
                acc = acc + _vln(ratio)
            return acc

        def group_body(g, acc):
            c0 = g * nb
            for b in range(nb):
                c = c0 + b
                cp_pos(c, b).wait()
                cp_neg(c, b).wait()
                acc = compute_chunk(c, b, acc)
                nxt = c + nb

                @pl.when(nxt < nchunk)
                def _():
                    cp_pos(nxt, b).start()
                    cp_neg(nxt, b).start()
            return acc

        acc = lax.fori_loop(0, nchunk // nb, group_body, zero)
        accv[...] = acc
        pltpu.sync_copy(accv, out_hbm.at[wid])

    partial = sc_kernel(z_all, knn_pad, neg_flat, cell32)
    return jnp.sum(partial) / (16.0 * batch)


# speedup vs baseline: 2.6211x; 1.0067x over previous
import functools

import jax
import jax.numpy as jnp
from jax import lax
from jax.experimental import pallas as pl
from jax.experimental.pallas import tpu as pltpu
from jax.experimental.pallas import tpu_sc as plsc

_LN2 = 0.6931471805599453


def _vsqrt(v):
    i = plsc.bitcast(v, jnp.int32)
    y = plsc.bitcast(jnp.int32(0x5F3759DF) - (i >> 1), jnp.float32)
    for _ in range(3):
        y = y * (1.5 - 0.5 * v * y * y)
    return v * y


def _vln(r):
    i = plsc.bitcast(r, jnp.int32)
    ex = ((i >> 23) & 0xFF) - 127
    f = plsc.bitcast((i & 0x007FFFFF) | 0x3F800000, jnp.float32)
    y = ex.astype(jnp.float32) * _LN2 + (f - 1.0)
    for _ in range(3):
        y = y - 1.0 + r * jnp.exp(-y)
    return y


def kernel(z_all, pre_knn_indices, cell_indices):
    n_cells, dim = z_all.shape
    batch = cell_indices.shape[0]
    k = pre_knn_indices.shape[1]
    kp = 32

    info = plsc.get_sparse_core_info()
    nw = info.num_cores * info.num_subcores
    bpw = batch // nw
    ch = 4
    nchunk = bpw // ch
    nidx = ch * kp

    cell32 = cell_indices.astype(jnp.int32)
    knn32 = pre_knn_indices.astype(jnp.int32)
    knn_pad = jnp.concatenate([knn32, knn32[:, :kp - k]], axis=1)
    neg = jax.random.randint(jax.random.key(1234), (batch, k), 0, n_cells,
                             dtype=jnp.int32)
    neg_flat = jnp.concatenate([neg, neg[:, :kp - k]],
                               axis=1).reshape(batch * kp)

    mesh = plsc.VectorSubcoreMesh(core_axis_name="c", subcore_axis_name="s")

    @functools.partial(
        pl.kernel,
        out_type=jax.ShapeDtypeStruct((nw, 16), jnp.float32),
        mesh=mesh,
        compiler_params=pltpu.CompilerParams(needs_layout_passes=False,
                                             use_tc_tiling_on_sc=False),
        scratch_types=[
            pltpu.VMEM((bpw,), jnp.int32),
            pltpu.VMEM((bpw, dim), jnp.float32),
            pltpu.VMEM((bpw, kp), jnp.int32),
            pltpu.VMEM((bpw * kp,), jnp.int32),
            pltpu.VMEM((bpw * kp,), jnp.int32),
            pltpu.VMEM((4, nidx, dim), jnp.float32),
            pltpu.VMEM((4, nidx, dim), jnp.float32),
            pltpu.VMEM((16,), jnp.float32),
            pltpu.SemaphoreType.DMA,
            pltpu.SemaphoreType.DMA,
            pltpu.SemaphoreType.DMA,
            pltpu.SemaphoreType.DMA,
            pltpu.SemaphoreType.DMA,
            pltpu.SemaphoreType.DMA,
            pltpu.SemaphoreType.DMA,
            pltpu.SemaphoreType.DMA,
            pltpu.SemaphoreType.DMA,
        ],
    )
    def sc_kernel(z_hbm, knn_hbm, neg_hbm, cell_hbm, out_hbm,
                  cidx, zi, nnp, nnf, ngf, posb, negb, accv, sem,
                  ps0, ps1, ps2, ps3, ns0, ns1, ns2, ns3):
        wid = lax.axis_index("s") * info.num_cores + lax.axis_index("c")
        base = pl.multiple_of(wid * bpw, bpw)
        pltpu.sync_copy(cell_hbm.at[pl.ds(base, bpw)], cidx)
        cp_zi = pltpu.make_async_copy(z_hbm.at[cidx], zi, sem)
        cp_nn = pltpu.make_async_copy(knn_hbm.at[cidx], nnp, ns0)
        cp_zi.start()
        cp_nn.start()
        fbase = pl.multiple_of(wid * (bpw * kp), bpw * kp)
        pltpu.sync_copy(neg_hbm.at[pl.ds(fbase, bpw * kp)], ngf)
        cp_nn.wait()

        def rp_body(e, carry):
            nnf[pl.ds(e * kp, 16)] = nnp[e, pl.ds(0, 16)]
            nnf[pl.ds(e * kp + 16, 16)] = nnp[e, pl.ds(16, 16)]
            return carry

        lax.fori_loop(0, bpw, rp_body, 0)

        lanes = lax.iota(jnp.int32, 16)
        zero = jnp.zeros((16,), jnp.float32)
        psems = [ps0, ps1, ps2, ps3]
        nsems = [ns0, ns1, ns2, ns3]
        nb = 4

        def cp_pos(c, b):
            i0 = pl.multiple_of(c * nidx, nidx)
            return pltpu.make_async_copy(
                z_hbm.at[nnf.at[pl.ds(i0, nidx)]], posb.at[b], psems[b])

        def cp_neg(c, b):
            i0 = pl.multiple_of(c * nidx, nidx)
            return pltpu.make_async_copy(
                z_hbm.at[ngf.at[pl.ds(i0, nidx)]], negb.at[b], nsems[b])

        for b in range(nb):
            cp_pos(b, b).start()
            cp_neg(b, b).start()
        cp_zi.wait()

        dup = 16 - (kp - k)

        def compute_chunk(c, b, acc):
            bv = jnp.full((16,), b, jnp.int32)
            for ee in range(ch):
                e = c * ch + ee
                r0 = lanes + (ee * kp)
                r1 = r0 + 16

                def dbody(qq, accs):
                    a0, a1, a2, a3 = accs
                    zq = zi[e, pl.ds(qq * 16, 16)]
                    for r in range(16):
                        zv = zq[r]
                        cols = jnp.full((16,), qq * 16 + r, jnp.int32)
                        t0 = plsc.load_gather(posb, [bv, r0, cols]) - zv
                        t1 = plsc.load_gather(posb, [bv, r1, cols]) - zv
                        t2 = plsc.load_gather(negb, [bv, r0, cols]) - zv
                        t3 = plsc.load_gather(negb, [bv, r1, cols]) - zv
                        a0 = a0 + t0 * t0
                        a1 = a1 + t1 * t1
                        a2 = a2 + t2 * t2
                        a3 = a3 + t3 * t3
                    return (a0, a1, a2, a3)

                p0, p1, q0, q1 = lax.fori_loop(
                    0, dim // 16, dbody, (zero, zero, zero, zero))
                p1 = jnp.where(lanes < dup, p1, 1e30)
                q1 = jnp.where(lanes < dup, q1, 1e30)
                d0 = _vsqrt(p0)
                d1 = _vsqrt(p1)
                d2 = _vsqrt(q0)
                d3 = _vsqrt(q1)
                m = jnp.min(jnp.minimum(jnp.minimum(d0, d1),
                                        jnp.minimum(d2, d3)))
                mv = jnp.full((16,), m, jnp.float32)
                e0v = jnp.exp(mv - d0)
                e1v = jnp.exp(mv - d1)
                e2v = jnp.exp(mv - d2)
                e3v = jnp.exp(mv - d3)
                sp = jnp.full((16,), jnp.sum(e0v + e1v), jnp.float32)
                st = sp + jnp.full((16,), jnp.sum(e2v + e3v), jnp.float32)
                ratio = st / (sp + 1e-8 * st)
                acc = acc + _vln(ratio)
            return acc

        def group_body(g, acc):
            c0 = g * nb
            for b in range(nb):
                c = c0 + b
                cp_pos(c, b).wait()
                cp_neg(c, b).wait()
                acc = compute_chunk(c, b, acc)
                nxt = c + nb

                @pl.when(nxt < nchunk)
                def _():
                    cp_pos(nxt, b).start()
                    cp_neg(nxt, b).start()
            return acc

        acc = lax.fori_loop(0, nchunk // nb, group_body, zero)
        accv[...] = acc
        pltpu.sync_copy(accv, out_hbm.at[wid])

    partial = sc_kernel(z_all, knn_pad, neg_flat, cell32)
    return jnp.sum(partial) / (16.0 * batch)
